# baseline (speedup 1.0000x reference)
import jax
import jax.numpy as jnp
from jax import lax
from jax.experimental import pallas as pl
from jax.experimental.pallas import tpu as pltpu
from jax.experimental.pallas import tpu_sc as plsc

_B, _S, _D = 4, 4096, 1024
_NW = 32
_ELEMS_W = (_B * _S * _D) // _NW
_CHE = 16 * _D
_NCHUNK = _ELEMS_W // _CHE


def _sc_body(x_hbm, pe_hbm, out_hbm, xb, pb):
    c = lax.axis_index("c")
    s = lax.axis_index("s")
    wid = s * 2 + c
    base = wid * _ELEMS_W
    pe_base = lax.rem(wid, 8) * _ELEMS_W

    def chunk(i, carry):
        off = base + i * _CHE
        poff = pe_base + i * _CHE
        pltpu.sync_copy(x_hbm.at[pl.ds(off, _CHE)], xb)
        pltpu.sync_copy(pe_hbm.at[pl.ds(poff, _CHE)], pb)

        @plsc.parallel_loop(0, _CHE, step=16, unroll=8)
        def _add(j):
            xb[pl.ds(j, 16)] = xb[pl.ds(j, 16)] + pb[pl.ds(j, 16)]

        pltpu.sync_copy(xb, out_hbm.at[pl.ds(off, _CHE)])
        return carry

    lax.fori_loop(0, _NCHUNK, chunk, 0)


def kernel(x, pe_table):
    xf = x.reshape(_B * _S * _D)
    pef = pe_table[:_S].reshape(_S * _D)
    mesh = plsc.VectorSubcoreMesh(core_axis_name="c", subcore_axis_name="s")
    out = pl.kernel(
        _sc_body,
        out_type=jax.ShapeDtypeStruct((_B * _S * _D,), jnp.float32),
        mesh=mesh,
        scratch_types=[
            pltpu.VMEM((_CHE,), jnp.float32),
            pltpu.VMEM((_CHE,), jnp.float32),
        ],
    )(xf, pef)
    return out.reshape(_B, _S, _D)


# speedup vs baseline: 1.2905x; 1.2905x over previous
import jax
import jax.numpy as jnp
from jax import lax
from jax.experimental import pallas as pl
from jax.experimental.pallas import tpu as pltpu
from jax.experimental.pallas import tpu_sc as plsc

_B, _S, _D = 4, 4096, 1024
_NW = 32
_ELEMS_W = (_B * _S * _D) // _NW
_CHE = 16 * _D
_NCHUNK = _ELEMS_W // _CHE
_NSLOT = 3


def _sc_body(x_hbm, pe_hbm, out_hbm, *refs):
    xbs, pbs = refs[0:_NSLOT], refs[_NSLOT:2 * _NSLOT]
    lsems, ssems = refs[2 * _NSLOT:3 * _NSLOT], refs[3 * _NSLOT:4 * _NSLOT]

    c = lax.axis_index("c")
    s = lax.axis_index("s")
    wid = s * 2 + c
    base = wid * _ELEMS_W
    pe_base = lax.rem(wid, 8) * _ELEMS_W

    loads, stores = {}, {}

    def start_load(i):
        sl = i % _NSLOT
        cx = pltpu.make_async_copy(
            x_hbm.at[pl.ds(base + i * _CHE, _CHE)], xbs[sl], lsems[sl])
        cp = pltpu.make_async_copy(
            pe_hbm.at[pl.ds(pe_base + i * _CHE, _CHE)], pbs[sl], lsems[sl])
        cx.start()
        cp.start()
        loads[i] = (cx, cp)

    def start_store(i):
        sl = i % _NSLOT
        cs = pltpu.make_async_copy(
            xbs[sl], out_hbm.at[pl.ds(base + i * _CHE, _CHE)], ssems[sl])
        cs.start()
        stores[i] = cs

    start_load(0)
    start_load(1)
    for i in range(_NCHUNK):
        if i >= 1:
            stores.pop(i - 1).wait()
        if i + 2 < _NCHUNK:
            start_load(i + 2)
        cx, cp = loads.pop(i)
        cx.wait()
        cp.wait()
        sl = i % _NSLOT

        @plsc.parallel_loop(0, _CHE, step=16, unroll=8)
        def _add(j, xb=xbs[sl], pb=pbs[sl]):
            xb[pl.ds(j, 16)] = xb[pl.ds(j, 16)] + pb[pl.ds(j, 16)]

        start_store(i)
    stores.pop(_NCHUNK - 1).wait()


def kernel(x, pe_table):
    xf = x.reshape(_B * _S * _D)
    pef = pe_table[:_S].reshape(_S * _D)
    mesh = plsc.VectorSubcoreMesh(core_axis_name="c", subcore_axis_name="s")
    out = pl.kernel(
        _sc_body,
        out_type=jax.ShapeDtypeStruct((_B * _S * _D,), jnp.float32),
        mesh=mesh,
        scratch_types=(
            [pltpu.VMEM((_CHE,), jnp.float32) for _ in range(2 * _NSLOT)]
            + [pltpu.SemaphoreType.DMA for _ in range(2 * _NSLOT)]
        ),
    )(xf, pef)
    return out.reshape(_B, _S, _D)
